# baseline (speedup 1.0000x reference)
import functools

import jax
import jax.numpy as jnp
from jax import lax
from jax.experimental import pallas as pl
from jax.experimental.pallas import tpu as pltpu
from jax.experimental.pallas import tpu_sc as plsc

NC = 2
NS = 16
K = 128
NBUF = 4


def _ceil_to(x, m):
  return ((x + m - 1) // m) * m


def _embed_body(f_ref, w_ref, b_ref, o_ref):
  h = jnp.dot(f_ref[...], w_ref[...], preferred_element_type=jnp.float32)
  h = jnp.maximum(h + b_ref[...], 0.0)
  o_ref[0] = h[:, :32]
  o_ref[1] = h[:, 32:]


def _embed(feat, w, b, bn):
  n = feat.shape[0]
  fi = feat.shape[1]
  return pl.pallas_call(
      _embed_body,
      grid=(n // bn,),
      in_specs=[
          pl.BlockSpec((bn, fi), lambda i: (i, 0)),
          pl.BlockSpec((fi, 64), lambda i: (0, 0)),
          pl.BlockSpec((1, 64), lambda i: (0, 0)),
      ],
      out_specs=pl.BlockSpec((2, bn, 32), lambda i: (0, i, 0)),
      out_shape=jax.ShapeDtypeStruct((2, n, 32), jnp.float32),
  )(feat, w, b.reshape(1, 64))


def _update_body(h_ref, m_ref, w_ref, b_ref, o_ref):
  h = jnp.concatenate([h_ref[0], h_ref[1]], axis=-1)
  m = jnp.concatenate([m_ref[0], m_ref[1]], axis=-1)
  o = jnp.dot(m, w_ref[...], preferred_element_type=jnp.float32)
  o = jnp.maximum(h + o + b_ref[...], 0.0)
  o_ref[0] = o[:, :32]
  o_ref[1] = o[:, 32:]


def _update(h, msgs, w, b, bn):
  n = h.shape[1]
  return pl.pallas_call(
      _update_body,
      grid=(n // bn,),
      in_specs=[
          pl.BlockSpec((2, bn, 32), lambda i: (0, i, 0)),
          pl.BlockSpec((2, bn, 32), lambda i: (0, i, 0)),
          pl.BlockSpec((64, 64), lambda i: (0, 0)),
          pl.BlockSpec((1, 64), lambda i: (0, 0)),
      ],
      out_specs=pl.BlockSpec((2, bn, 32), lambda i: (0, i, 0)),
      out_shape=jax.ShapeDtypeStruct((2, n, 32), jnp.float32),
  )(h, msgs, w, b.reshape(1, 64))


def _score_body(h_ref, m_ref, w_ref, b_ref, ws_ref, bs_ref, o_ref):
  h = jnp.concatenate([h_ref[0], h_ref[1]], axis=-1)
  m = jnp.concatenate([m_ref[0], m_ref[1]], axis=-1)
  o = jnp.dot(m, w_ref[...], preferred_element_type=jnp.float32)
  o = jnp.maximum(h + o + b_ref[...], 0.0)
  o_ref[...] = jnp.dot(o, ws_ref[...], preferred_element_type=jnp.float32) + bs_ref[...]


def _score(h, msgs, w, b, ws, bs, bn):
  n = h.shape[1]
  return pl.pallas_call(
      _score_body,
      grid=(n // bn,),
      in_specs=[
          pl.BlockSpec((2, bn, 32), lambda i: (0, i, 0)),
          pl.BlockSpec((2, bn, 32), lambda i: (0, i, 0)),
          pl.BlockSpec((64, 64), lambda i: (0, 0)),
          pl.BlockSpec((1, 64), lambda i: (0, 0)),
          pl.BlockSpec((64, 1), lambda i: (0, 0)),
          pl.BlockSpec((1, 1), lambda i: (0, 0)),
      ],
      out_specs=pl.BlockSpec((bn, 1), lambda i: (i, 0)),
      out_shape=jax.ShapeDtypeStruct((n, 1), jnp.float32),
  )(h, msgs, w, b.reshape(1, 64), ws, bs.reshape(1, 1))


D = 12
R = 6
G = 3


@functools.cache
def _make_sc_pass(n_src, n_dst_pad, nchunk_tot):
  del n_src
  nchunk_t = nchunk_tot // NS
  rows_per_tile = n_dst_pad // NS
  nz = rows_per_tile // K
  assert nchunk_t % D == 0 and nchunk_t > D
  mesh = plsc.VectorSubcoreMesh(core_axis_name="c", subcore_axis_name="s")

  @functools.partial(
      pl.kernel,
      out_type=jax.ShapeDtypeStruct((NC, n_dst_pad, 32), jnp.float32),
      mesh=mesh,
      scratch_types=[
          pltpu.VMEM_SHARED((n_dst_pad, 32), jnp.float32),
          pltpu.VMEM((D, 2, K), jnp.int32),
          pltpu.VMEM((R, K, 32), jnp.float32),
          [pltpu.SemaphoreType.DMA] * D,
          [pltpu.SemaphoreType.DMA] * R,
          [pltpu.SemaphoreType.DMA] * R,
      ],
      compiler_params=pltpu.CompilerParams(use_tc_tiling_on_sc=False),
  )
  def sc_pass(t_hbm, idx_hbm, out_hbm, accum, idx_v, rows_v, isem, gsem, ssem):
    c = lax.axis_index("c")
    s = lax.axis_index("s")
    row0 = s * nchunk_t

    def _gather(q, u):
      return pltpu.make_async_copy(
          t_hbm.at[c].at[idx_v.at[q].at[0]], rows_v.at[u], gsem[u])

    def _scatter(q, u):
      return pltpu.make_async_copy(
          rows_v.at[u], accum.at[idx_v.at[q].at[1]], ssem[u])

    def _idx_fetch(m, q):
      return pltpu.make_async_copy(idx_hbm.at[row0 + m], idx_v.at[q], isem[q])

    def zf(i, carry):
      rows_v[0, i, pl.ds(0, 16)] = jnp.zeros((16,), jnp.float32)
      rows_v[0, i, pl.ds(16, 16)] = jnp.zeros((16,), jnp.float32)
      return carry
    lax.fori_loop(0, K, zf, 0)

    def zc(i, carry):
      pltpu.sync_copy(rows_v.at[0], accum.at[pl.ds((s * nz + i) * K, K)])
      return carry
    lax.fori_loop(0, nz, zc, 0)

    for q in range(D):
      _idx_fetch(q, q).start()
    for u in range(G):
      _idx_fetch(u, u).wait()
      _gather(u, u).start()

    plsc.subcore_barrier()

    def inner(jj, carry):
      base = jj * D
      for k in range(D):
        i = base + k
        u = k % R
        q = k
        u3 = (k + G) % R
        q3 = (k + G) % D
        qr = (k - G) % D
        _gather(q, u).wait()
        _scatter(q, u).start(add=True)

        @pl.when(i >= G)
        def _wait_old_scatter():
          _scatter(qr, u3).wait()

        @pl.when(i + G < nchunk_t)
        def _issue_gather():
          _idx_fetch(i + G, q3).wait()
          _gather(q3, u3).start()

        @pl.when((i >= G) & (i + D - G < nchunk_t))
        def _refetch_idx():
          _idx_fetch(i + D - G, qr).start()
      return carry
    lax.fori_loop(0, nchunk_t // D, inner, 0)

    for m in range(nchunk_t - G, nchunk_t):
      _scatter(m % D, m % R).wait()

    plsc.subcore_barrier()
    pltpu.sync_copy(
        accum.at[pl.ds(s * rows_per_tile, rows_per_tile)],
        out_hbm.at[c].at[pl.ds(s * rows_per_tile, rows_per_tile)])

  return sc_pass


def kernel(var_feat, constr_feat, edge_index_var_to_constr,
           W_var, b_var, W_constr, b_constr,
           W_v2c, b_v2c, W_c2v, b_c2v, W_score, b_score):
  v = var_feat.shape[0]
  cn = constr_feat.shape[0]
  e = edge_index_var_to_constr.shape[1]

  v_pad = _ceil_to(v + 1, NS * K)
  c_pad = _ceil_to(cn + 1, NS * K)
  e_pad = _ceil_to(e + 1, NS * K * D)
  nchunk_tot = e_pad // K

  eidx = edge_index_var_to_constr.astype(jnp.int32)
  vidx, cidx = eidx[0], eidx[1]
  npad = e_pad - e
  ar = jnp.arange(npad, dtype=jnp.int32)
  sidx_v2c = jnp.concatenate([vidx, ar % v]).reshape(nchunk_tot, 1, K)
  didx_v2c = jnp.concatenate([cidx, cn + ar % (c_pad - cn)]).reshape(nchunk_tot, 1, K)
  sidx_c2v = jnp.concatenate([cidx, ar % cn]).reshape(nchunk_tot, 1, K)
  didx_c2v = jnp.concatenate([vidx, v + ar % (v_pad - v)]).reshape(nchunk_tot, 1, K)
  idx_v2c = jnp.concatenate([sidx_v2c, didx_v2c], axis=1)
  idx_c2v = jnp.concatenate([sidx_c2v, didx_c2v], axis=1)

  v2c = _make_sc_pass(v, c_pad, nchunk_tot)
  c2v = _make_sc_pass(cn, v_pad, nchunk_tot)

  h_var = _embed(var_feat, W_var, b_var, 1000)
  h_constr = _embed(constr_feat, W_constr, b_constr, 1000)

  rounds = 3
  for r in range(rounds):
    msgs_c = v2c(h_var, idx_v2c)
    h_constr = _update(h_constr, msgs_c, W_v2c, b_v2c, 1000)
    msgs_v = c2v(h_constr, idx_c2v)
    if r < rounds - 1:
      h_var = _update(h_var, msgs_v, W_c2v, b_c2v, 1000)
    else:
      scores = _score(h_var, msgs_v, W_c2v, b_c2v, W_score, b_score, 1000)

  return scores.reshape(-1)


# speedup vs baseline: 1.5308x; 1.5308x over previous
import functools

import jax
import jax.numpy as jnp
from jax import lax
from jax.experimental import pallas as pl
from jax.experimental.pallas import tpu as pltpu
from jax.experimental.pallas import tpu_sc as plsc

NC = 2
NS = 16
K = 128
NBUF = 4
D = 8


def _ceil_to(x, m):
  return ((x + m - 1) // m) * m


def _embed_body(f_ref, w_ref, b_ref, o_ref):
  h = jnp.dot(f_ref[...], w_ref[...], preferred_element_type=jnp.float32)
  o_ref[...] = jnp.maximum(h + b_ref[...], 0.0)


def _embed(feat2, w2, b2, bnp):
  n2, fi2 = feat2.shape
  return pl.pallas_call(
      _embed_body,
      grid=(n2 // bnp,),
      in_specs=[
          pl.BlockSpec((bnp, fi2), lambda i: (i, 0)),
          pl.BlockSpec((fi2, 128), lambda i: (0, 0)),
          pl.BlockSpec((1, 128), lambda i: (0, 0)),
      ],
      out_specs=pl.BlockSpec((bnp, 128), lambda i: (i, 0)),
      out_shape=jax.ShapeDtypeStruct((n2, 128), jnp.float32),
  )(feat2, w2, b2.reshape(1, 128))


def _update_body(h_ref, m_ref, w_ref, b_ref, o_ref):
  o = jnp.dot(m_ref[...], w_ref[...], preferred_element_type=jnp.float32)
  o_ref[...] = jnp.maximum(h_ref[...] + o + b_ref[...], 0.0)


def _update(h2, msgs2, w2, b2, bnp):
  n2 = h2.shape[0]
  return pl.pallas_call(
      _update_body,
      grid=(n2 // bnp,),
      in_specs=[
          pl.BlockSpec((bnp, 128), lambda i: (i, 0)),
          pl.BlockSpec((bnp, 128), lambda i: (i, 0)),
          pl.BlockSpec((128, 128), lambda i: (0, 0)),
          pl.BlockSpec((1, 128), lambda i: (0, 0)),
      ],
      out_specs=pl.BlockSpec((bnp, 128), lambda i: (i, 0)),
      out_shape=jax.ShapeDtypeStruct((n2, 128), jnp.float32),
  )(h2, msgs2, w2, b2.reshape(1, 128))


def _score_body(h_ref, m_ref, w_ref, b_ref, ws_ref, bs_ref, o_ref):
  o = jnp.dot(m_ref[...], w_ref[...], preferred_element_type=jnp.float32)
  o = jnp.maximum(h_ref[...] + o + b_ref[...], 0.0)
  o_ref[...] = jnp.dot(o, ws_ref[...], preferred_element_type=jnp.float32) + bs_ref[...]


def _score(h2, msgs2, w2, b2, ws2, bs2, bnp):
  n2 = h2.shape[0]
  return pl.pallas_call(
      _score_body,
      grid=(n2 // bnp,),
      in_specs=[
          pl.BlockSpec((bnp, 128), lambda i: (i, 0)),
          pl.BlockSpec((bnp, 128), lambda i: (i, 0)),
          pl.BlockSpec((128, 128), lambda i: (0, 0)),
          pl.BlockSpec((1, 128), lambda i: (0, 0)),
          pl.BlockSpec((128, 2), lambda i: (0, 0)),
          pl.BlockSpec((1, 2), lambda i: (0, 0)),
      ],
      out_specs=pl.BlockSpec((bnp, 2), lambda i: (i, 0)),
      out_shape=jax.ShapeDtypeStruct((n2, 2), jnp.float32),
  )(h2, msgs2, w2, b2.reshape(1, 128), ws2, bs2.reshape(1, 2))


@functools.cache
def _make_sc_pass(n_src2, n_dst, n_dst_pad, nchunk_tot):
  del n_src2
  nchunk_t = nchunk_tot // NS
  rows_per_tile = n_dst_pad // NS
  nz = rows_per_tile // K
  dr0 = n_dst // NS
  dr_last = n_dst - dr0 * (NS - 1)
  assert nchunk_t % D == 0
  mesh = plsc.VectorSubcoreMesh(core_axis_name="c", subcore_axis_name="s")

  @functools.partial(
      pl.kernel,
      out_type=jax.ShapeDtypeStruct((n_dst, 64), jnp.float32),
      mesh=mesh,
      scratch_types=[
          pltpu.VMEM_SHARED((n_dst_pad, 32), jnp.float32),
          pltpu.VMEM((D, 2, K), jnp.int32),
          pltpu.VMEM((NBUF, K, 32), jnp.float32),
          [pltpu.SemaphoreType.DMA] * D,
          [pltpu.SemaphoreType.DMA] * NBUF,
      ],
      compiler_params=pltpu.CompilerParams(use_tc_tiling_on_sc=False),
  )
  def sc_pass(t_hbm, idx_hbm, out_hbm, accum, idx_v, rows_v, isem, gsem):
    c = lax.axis_index("c")
    s = lax.axis_index("s")
    row0 = s * nchunk_t

    def zf(i, carry):
      rows_v[0, i, pl.ds(0, 16)] = jnp.zeros((16,), jnp.float32)
      rows_v[0, i, pl.ds(16, 16)] = jnp.zeros((16,), jnp.float32)
      return carry
    lax.fori_loop(0, K, zf, 0)

    def zc(i, carry):
      pltpu.sync_copy(rows_v.at[0], accum.at[pl.ds((s * nz + i) * K, K)])
      return carry
    lax.fori_loop(0, nz, zc, 0)

    for u in range(D):
      pltpu.async_copy(idx_hbm.at[c].at[row0 + u], idx_v.at[u], isem[u])
    for u in range(NBUF):
      pltpu.make_async_copy(
          idx_hbm.at[c].at[row0 + u], idx_v.at[u], isem[u]).wait()
      pltpu.async_copy(t_hbm.at[idx_v.at[u].at[0]], rows_v.at[u], gsem[u])

    plsc.subcore_barrier()

    def inner(jj, carry):
      base = jj * D
      for u in range(D):
        j = base + u
        b = u % NBUF
        un = (u + NBUF) % D
        pltpu.make_async_copy(
            t_hbm.at[idx_v.at[u].at[0]], rows_v.at[b], gsem[b]).wait()
        pltpu.sync_copy(rows_v.at[b], accum.at[idx_v.at[u].at[1]], add=True)
        @pl.when(j + D < nchunk_t)
        def _refill():
          pltpu.async_copy(idx_hbm.at[c].at[row0 + j + D], idx_v.at[u], isem[u])
        @pl.when(j + NBUF < nchunk_t)
        def _issue():
          pltpu.make_async_copy(
              idx_hbm.at[c].at[row0 + j + NBUF], idx_v.at[un], isem[un]).wait()
          pltpu.async_copy(
              t_hbm.at[idx_v.at[un].at[0]], rows_v.at[b], gsem[b])
      return carry
    lax.fori_loop(0, nchunk_t // D, inner, 0)

    plsc.subcore_barrier()

    def _drain(sl):
      @pl.when(c == 0)
      def _d0():
        pltpu.sync_copy(accum.at[sl], out_hbm.at[sl, pl.ds(0, 32)])

      @pl.when(c == 1)
      def _d1():
        pltpu.sync_copy(accum.at[sl], out_hbm.at[sl, pl.ds(32, 32)])

    if dr0 * NS == n_dst:
      _drain(pl.ds(s * dr0, dr0))
    else:
      @pl.when(s < NS - 1)
      def _not_last():
        _drain(pl.ds(s * dr0, dr0))

      @pl.when(s == NS - 1)
      def _last():
        _drain(pl.ds((NS - 1) * dr0, dr_last))

  return sc_pass


def kernel(var_feat, constr_feat, edge_index_var_to_constr,
           W_var, b_var, W_constr, b_constr,
           W_v2c, b_v2c, W_c2v, b_c2v, W_score, b_score):
  v = var_feat.shape[0]
  cn = constr_feat.shape[0]
  e = edge_index_var_to_constr.shape[1]

  v_pad = _ceil_to(v + 1, NS * K)
  c_pad = _ceil_to(cn + 1, NS * K)
  e_pad = _ceil_to(e + 1, NS * K * D)
  nchunk_tot = e_pad // K

  eidx = edge_index_var_to_constr.astype(jnp.int32)
  vidx, cidx = eidx[0], eidx[1]
  npad = e_pad - e
  ar = jnp.arange(npad, dtype=jnp.int32)
  core = jnp.arange(NC, dtype=jnp.int32).reshape(NC, 1, 1, 1)

  def _mk_idx(sidx, didx):
    s2 = (2 * sidx).reshape(1, nchunk_tot, 1, K) + core
    d2 = jnp.broadcast_to(didx.reshape(1, nchunk_tot, 1, K), (NC, nchunk_tot, 1, K))
    return jnp.concatenate([s2, d2], axis=2)

  idx_v2c = _mk_idx(jnp.concatenate([vidx, ar % v]),
                    jnp.concatenate([cidx, cn + ar % (c_pad - cn)]))
  idx_c2v = _mk_idx(jnp.concatenate([cidx, ar % cn]),
                    jnp.concatenate([vidx, v + ar % (v_pad - v)]))

  v2c = _make_sc_pass(2 * v, cn, c_pad, nchunk_tot)
  c2v = _make_sc_pass(2 * cn, v, v_pad, nchunk_tot)

  eye2 = jnp.eye(2, dtype=jnp.float32)
  w_var2 = jnp.kron(eye2, W_var)
  w_constr2 = jnp.kron(eye2, W_constr)
  w_v2c2 = jnp.kron(eye2, W_v2c)
  w_c2v2 = jnp.kron(eye2, W_c2v)
  ws2 = jnp.kron(eye2, W_score)
  b_var2 = jnp.tile(b_var, 2)
  b_constr2 = jnp.tile(b_constr, 2)
  b_v2c2 = jnp.tile(b_v2c, 2)
  b_c2v2 = jnp.tile(b_c2v, 2)
  bs2 = jnp.tile(b_score, 2)

  h_var = _embed(var_feat.reshape(v // 2, 256), w_var2, b_var2, 1000)
  h_constr = _embed(constr_feat.reshape(cn // 2, 256), w_constr2, b_constr2, cn // 2)

  rounds = 3
  for r in range(rounds):
    msgs_c = v2c(h_var.reshape(2 * v, 32), idx_v2c)
    h_constr = _update(h_constr, msgs_c.reshape(cn // 2, 128),
                       w_v2c2, b_v2c2, cn // 2)
    msgs_v = c2v(h_constr.reshape(2 * cn, 32), idx_c2v)
    if r < rounds - 1:
      h_var = _update(h_var, msgs_v.reshape(v // 2, 128),
                      w_c2v2, b_c2v2, 1000)
    else:
      scores = _score(h_var, msgs_v.reshape(v // 2, 128),
                      w_c2v2, b_c2v2, ws2, bs2, 1000)

  return scores.reshape(-1)


# speedup vs baseline: 1.6364x; 1.0690x over previous
import functools

import jax
import jax.numpy as jnp
from jax import lax
from jax.experimental import pallas as pl
from jax.experimental.pallas import tpu as pltpu
from jax.experimental.pallas import tpu_sc as plsc

NC = 2
NS = 16
K = 128
NBUF = 4
D = 8


def _ceil_to(x, m):
  return ((x + m - 1) // m) * m


def _embed_body(f_ref, w_ref, b_ref, o_ref):
  h = jnp.dot(f_ref[...], w_ref[...], preferred_element_type=jnp.float32)
  o_ref[...] = jnp.maximum(h + b_ref[...], 0.0)


def _embed(feat2, w2, b2, bnp):
  n2, fi2 = feat2.shape
  return pl.pallas_call(
      _embed_body,
      grid=(n2 // bnp,),
      in_specs=[
          pl.BlockSpec((bnp, fi2), lambda i: (i, 0)),
          pl.BlockSpec((fi2, 128), lambda i: (0, 0)),
          pl.BlockSpec((1, 128), lambda i: (0, 0)),
      ],
      out_specs=pl.BlockSpec((bnp, 128), lambda i: (i, 0)),
      out_shape=jax.ShapeDtypeStruct((n2, 128), jnp.float32),
  )(feat2, w2, b2.reshape(1, 128))


def _update_body(h_ref, m_ref, w_ref, b_ref, o_ref):
  o = jnp.dot(m_ref[...], w_ref[...], preferred_element_type=jnp.float32)
  o_ref[...] = jnp.maximum(h_ref[...] + o + b_ref[...], 0.0)


def _update(h2, msgs2, w2, b2, bnp):
  n2 = h2.shape[0]
  return pl.pallas_call(
      _update_body,
      grid=(n2 // bnp,),
      in_specs=[
          pl.BlockSpec((bnp, 128), lambda i: (i, 0)),
          pl.BlockSpec((bnp, 128), lambda i: (i, 0)),
          pl.BlockSpec((128, 128), lambda i: (0, 0)),
          pl.BlockSpec((1, 128), lambda i: (0, 0)),
      ],
      out_specs=pl.BlockSpec((bnp, 128), lambda i: (i, 0)),
      out_shape=jax.ShapeDtypeStruct((n2, 128), jnp.float32),
  )(h2, msgs2, w2, b2.reshape(1, 128))


def _score_body(h_ref, m_ref, w_ref, b_ref, ws_ref, bs_ref, o_ref):
  o = jnp.dot(m_ref[...], w_ref[...], preferred_element_type=jnp.float32)
  o = jnp.maximum(h_ref[...] + o + b_ref[...], 0.0)
  o_ref[...] = jnp.dot(o, ws_ref[...], preferred_element_type=jnp.float32) + bs_ref[...]


def _score(h2, msgs2, w2, b2, ws2, bs2, bnp):
  n2 = h2.shape[0]
  return pl.pallas_call(
      _score_body,
      grid=(n2 // bnp,),
      in_specs=[
          pl.BlockSpec((bnp, 128), lambda i: (i, 0)),
          pl.BlockSpec((bnp, 128), lambda i: (i, 0)),
          pl.BlockSpec((128, 128), lambda i: (0, 0)),
          pl.BlockSpec((1, 128), lambda i: (0, 0)),
          pl.BlockSpec((128, 2), lambda i: (0, 0)),
          pl.BlockSpec((1, 2), lambda i: (0, 0)),
      ],
      out_specs=pl.BlockSpec((bnp, 2), lambda i: (i, 0)),
      out_shape=jax.ShapeDtypeStruct((n2, 2), jnp.float32),
  )(h2, msgs2, w2, b2.reshape(1, 128), ws2, bs2.reshape(1, 2))


@functools.cache
def _make_sc_pass(n_src2, n_dst, n_dst_pad, nchunk_tot):
  nchunk_t = nchunk_tot // NS
  rows_per_tile = n_dst_pad // NS
  dr0 = n_dst // NS
  dr_last = n_dst - dr0 * (NS - 1)
  assert nchunk_t % D == 0
  mesh = plsc.VectorSubcoreMesh(core_axis_name="c", subcore_axis_name="s")

  @functools.partial(
      pl.kernel,
      out_type=jax.ShapeDtypeStruct((n_dst, 64), jnp.float32),
      mesh=mesh,
      scratch_types=[
          pltpu.VMEM_SHARED((n_dst_pad, 32), jnp.float32),
          pltpu.VMEM((D, 2, K), jnp.int32),
          pltpu.VMEM((NBUF, K, 32), jnp.float32),
          [pltpu.SemaphoreType.DMA] * D,
          [pltpu.SemaphoreType.DMA] * NBUF,
      ],
      compiler_params=pltpu.CompilerParams(use_tc_tiling_on_sc=False),
  )
  def sc_pass(t_hbm, idx_hbm, zeros_hbm, out_hbm, accum, idx_v, rows_v,
              isem, gsem):
    c = lax.axis_index("c")
    s = lax.axis_index("s")
    row0 = s * nchunk_t
    t_half = t_hbm.at[pl.ds(c, n_src2 - 1)]

    pltpu.sync_copy(zeros_hbm.at[pl.ds(s * rows_per_tile, rows_per_tile)],
                    accum.at[pl.ds(s * rows_per_tile, rows_per_tile)])

    for u in range(D):
      pltpu.async_copy(idx_hbm.at[row0 + u], idx_v.at[u], isem[u])
    for u in range(NBUF):
      pltpu.make_async_copy(
          idx_hbm.at[row0 + u], idx_v.at[u], isem[u]).wait()
      pltpu.async_copy(t_half.at[idx_v.at[u].at[0]], rows_v.at[u], gsem[u])

    plsc.subcore_barrier()

    def inner(jj, carry):
      base = jj * D
      for u in range(D):
        j = base + u
        b = u % NBUF
        un = (u + NBUF) % D
        pltpu.make_async_copy(
            t_half.at[idx_v.at[u].at[0]], rows_v.at[b], gsem[b]).wait()
        pltpu.sync_copy(rows_v.at[b], accum.at[idx_v.at[u].at[1]], add=True)
        @pl.when(j + D < nchunk_t)
        def _refill():
          pltpu.async_copy(idx_hbm.at[row0 + j + D], idx_v.at[u], isem[u])
        @pl.when(j + NBUF < nchunk_t)
        def _issue():
          pltpu.make_async_copy(
              idx_hbm.at[row0 + j + NBUF], idx_v.at[un], isem[un]).wait()
          pltpu.async_copy(
              t_half.at[idx_v.at[un].at[0]], rows_v.at[b], gsem[b])
      return carry
    lax.fori_loop(0, nchunk_t // D, inner, 0)

    plsc.subcore_barrier()

    def _drain(sl):
      @pl.when(c == 0)
      def _d0():
        pltpu.sync_copy(accum.at[sl], out_hbm.at[sl, pl.ds(0, 32)])

      @pl.when(c == 1)
      def _d1():
        pltpu.sync_copy(accum.at[sl], out_hbm.at[sl, pl.ds(32, 32)])

    if dr0 * NS == n_dst:
      _drain(pl.ds(s * dr0, dr0))
    else:
      @pl.when(s < NS - 1)
      def _not_last():
        _drain(pl.ds(s * dr0, dr0))

      @pl.when(s == NS - 1)
      def _last():
        _drain(pl.ds((NS - 1) * dr0, dr_last))

  return sc_pass


def kernel(var_feat, constr_feat, edge_index_var_to_constr,
           W_var, b_var, W_constr, b_constr,
           W_v2c, b_v2c, W_c2v, b_c2v, W_score, b_score):
  v = var_feat.shape[0]
  cn = constr_feat.shape[0]
  e = edge_index_var_to_constr.shape[1]

  v_pad = _ceil_to(v + 1, NS * K)
  c_pad = _ceil_to(cn + 1, NS * K)
  e_pad = _ceil_to(e + 1, NS * K * D)
  nchunk_tot = e_pad // K

  eidx = edge_index_var_to_constr.astype(jnp.int32)
  vidx, cidx = eidx[0], eidx[1]
  npad = e_pad - e
  ar = jnp.arange(npad, dtype=jnp.int32)
  def _mk_idx(sidx, didx):
    s2 = (2 * sidx).reshape(nchunk_tot, 1, K)
    d2 = didx.reshape(nchunk_tot, 1, K)
    return jnp.concatenate([s2, d2], axis=1)

  idx_v2c = _mk_idx(jnp.concatenate([vidx, ar % v]),
                    jnp.concatenate([cidx, cn + ar % (c_pad - cn)]))
  idx_c2v = _mk_idx(jnp.concatenate([cidx, ar % cn]),
                    jnp.concatenate([vidx, v + ar % (v_pad - v)]))

  v2c = _make_sc_pass(2 * v, cn, c_pad, nchunk_tot)
  c2v = _make_sc_pass(2 * cn, v, v_pad, nchunk_tot)

  eye2 = jnp.eye(2, dtype=jnp.float32)
  w_var2 = jnp.kron(eye2, W_var)
  w_constr2 = jnp.kron(eye2, W_constr)
  w_v2c2 = jnp.kron(eye2, W_v2c)
  w_c2v2 = jnp.kron(eye2, W_c2v)
  ws2 = jnp.kron(eye2, W_score)
  b_var2 = jnp.tile(b_var, 2)
  b_constr2 = jnp.tile(b_constr, 2)
  b_v2c2 = jnp.tile(b_v2c, 2)
  b_c2v2 = jnp.tile(b_c2v, 2)
  bs2 = jnp.tile(b_score, 2)

  zeros = jnp.zeros((v_pad, 32), jnp.float32)

  h_var = _embed(var_feat.reshape(v // 2, 256), w_var2, b_var2, 5000)
  h_constr = _embed(constr_feat.reshape(cn // 2, 256), w_constr2, b_constr2, cn // 2)

  rounds = 3
  for r in range(rounds):
    msgs_c = v2c(h_var.reshape(2 * v, 32), idx_v2c, zeros)
    h_constr = _update(h_constr, msgs_c.reshape(cn // 2, 128),
                       w_v2c2, b_v2c2, cn // 2)
    msgs_v = c2v(h_constr.reshape(2 * cn, 32), idx_c2v, zeros)
    if r < rounds - 1:
      h_var = _update(h_var, msgs_v.reshape(v // 2, 128),
                      w_c2v2, b_c2v2, 5000)
    else:
      scores = _score(h_var, msgs_v.reshape(v // 2, 128),
                      w_c2v2, b_c2v2, ws2, bs2, 5000)

  return scores.reshape(-1)


# speedup vs baseline: 1.6396x; 1.0019x over previous
import functools

import jax
import jax.numpy as jnp
from jax import lax
from jax.experimental import pallas as pl
from jax.experimental.pallas import tpu as pltpu
from jax.experimental.pallas import tpu_sc as plsc

NC = 2
NS = 16
K = 128
NBUF = 4
D = 8


def _ceil_to(x, m):
  return ((x + m - 1) // m) * m


def _embed_body(f_ref, w_ref, b_ref, o_ref):
  h = jnp.dot(f_ref[...], w_ref[...], preferred_element_type=jnp.float32)
  o_ref[...] = jnp.maximum(h + b_ref[...], 0.0)


def _embed(feat2, w2, b2, bnp):
  n2, fi2 = feat2.shape
  return pl.pallas_call(
      _embed_body,
      grid=(n2 // bnp,),
      in_specs=[
          pl.BlockSpec((bnp, fi2), lambda i: (i, 0)),
          pl.BlockSpec((fi2, 128), lambda i: (0, 0)),
          pl.BlockSpec((1, 128), lambda i: (0, 0)),
      ],
      out_specs=pl.BlockSpec((bnp, 128), lambda i: (i, 0)),
      out_shape=jax.ShapeDtypeStruct((n2, 128), jnp.float32),
  )(feat2, w2, b2.reshape(1, 128))


def _update_body(h_ref, m_ref, w_ref, b_ref, o_ref):
  o = jnp.dot(m_ref[...], w_ref[...], preferred_element_type=jnp.float32)
  o_ref[...] = jnp.maximum(h_ref[...] + o + b_ref[...], 0.0)


def _update(h2, msgs2, w2, b2, bnp):
  n2 = h2.shape[0]
  return pl.pallas_call(
      _update_body,
      grid=(n2 // bnp,),
      in_specs=[
          pl.BlockSpec((bnp, 128), lambda i: (i, 0)),
          pl.BlockSpec((bnp, 128), lambda i: (i, 0)),
          pl.BlockSpec((128, 128), lambda i: (0, 0)),
          pl.BlockSpec((1, 128), lambda i: (0, 0)),
      ],
      out_specs=pl.BlockSpec((bnp, 128), lambda i: (i, 0)),
      out_shape=jax.ShapeDtypeStruct((n2, 128), jnp.float32),
  )(h2, msgs2, w2, b2.reshape(1, 128))


def _score_body(h_ref, m_ref, w_ref, b_ref, ws_ref, bs_ref, o_ref):
  o = jnp.dot(m_ref[...], w_ref[...], preferred_element_type=jnp.float32)
  o = jnp.maximum(h_ref[...] + o + b_ref[...], 0.0)
  o_ref[...] = jnp.dot(o, ws_ref[...], preferred_element_type=jnp.float32) + bs_ref[...]


def _score(h2, msgs2, w2, b2, ws2, bs2, bnp):
  n2 = h2.shape[0]
  return pl.pallas_call(
      _score_body,
      grid=(n2 // bnp,),
      in_specs=[
          pl.BlockSpec((bnp, 128), lambda i: (i, 0)),
          pl.BlockSpec((bnp, 128), lambda i: (i, 0)),
          pl.BlockSpec((128, 128), lambda i: (0, 0)),
          pl.BlockSpec((1, 128), lambda i: (0, 0)),
          pl.BlockSpec((128, 2), lambda i: (0, 0)),
          pl.BlockSpec((1, 2), lambda i: (0, 0)),
      ],
      out_specs=pl.BlockSpec((bnp, 2), lambda i: (i, 0)),
      out_shape=jax.ShapeDtypeStruct((n2, 2), jnp.float32),
  )(h2, msgs2, w2, b2.reshape(1, 128), ws2, bs2.reshape(1, 2))


@functools.cache
def _make_sc_pass(n_src2, n_dst, n_dst_pad, nchunk_tot):
  nchunk_t = nchunk_tot // NS
  rows_per_tile = n_dst_pad // NS
  dr0 = n_dst // NS
  dr_last = n_dst - dr0 * (NS - 1)
  assert nchunk_t % D == 0
  mesh = plsc.VectorSubcoreMesh(core_axis_name="c", subcore_axis_name="s")

  @functools.partial(
      pl.kernel,
      out_type=jax.ShapeDtypeStruct((n_dst, 64), jnp.float32),
      mesh=mesh,
      scratch_types=[
          pltpu.VMEM_SHARED((n_dst_pad, 32), jnp.float32),
          pltpu.VMEM((D, 2, K), jnp.int32),
          pltpu.VMEM((NBUF, K, 32), jnp.float32),
          [pltpu.SemaphoreType.DMA] * D,
          [pltpu.SemaphoreType.DMA] * NBUF,
      ],
      compiler_params=pltpu.CompilerParams(use_tc_tiling_on_sc=False),
  )
  def sc_pass(t_hbm, sidx_hbm, didx_hbm, zeros_hbm, out_hbm, accum, idx_v,
              rows_v, isem, gsem):
    c = lax.axis_index("c")
    s = lax.axis_index("s")
    row0 = s * nchunk_t

    def _ifetch(row, u):
      pltpu.async_copy(sidx_hbm.at[row], idx_v.at[u].at[0], isem[u])
      pltpu.async_copy(didx_hbm.at[row], idx_v.at[u].at[1], isem[u])

    def _iwait(row, u):
      pltpu.make_async_copy(sidx_hbm.at[row], idx_v.at[u].at[0], isem[u]).wait()
      pltpu.make_async_copy(didx_hbm.at[row], idx_v.at[u].at[1], isem[u]).wait()
    t_half = t_hbm.at[pl.ds(c, n_src2 - 1)]

    pltpu.sync_copy(zeros_hbm.at[pl.ds(s * rows_per_tile, rows_per_tile)],
                    accum.at[pl.ds(s * rows_per_tile, rows_per_tile)])

    for u in range(D):
      _ifetch(row0 + u, u)
    for u in range(NBUF):
      _iwait(row0 + u, u)
      pltpu.async_copy(t_half.at[idx_v.at[u].at[0]], rows_v.at[u], gsem[u])

    plsc.subcore_barrier()

    def inner(jj, carry):
      base = jj * D
      for u in range(D):
        j = base + u
        b = u % NBUF
        un = (u + NBUF) % D
        pltpu.make_async_copy(
            t_half.at[idx_v.at[u].at[0]], rows_v.at[b], gsem[b]).wait()
        pltpu.sync_copy(rows_v.at[b], accum.at[idx_v.at[u].at[1]], add=True)
        @pl.when(j + D < nchunk_t)
        def _refill():
          _ifetch(row0 + j + D, u)
        @pl.when(j + NBUF < nchunk_t)
        def _issue():
          _iwait(row0 + j + NBUF, un)
          pltpu.async_copy(
              t_half.at[idx_v.at[un].at[0]], rows_v.at[b], gsem[b])
      return carry
    lax.fori_loop(0, nchunk_t // D, inner, 0)

    plsc.subcore_barrier()

    def _drain(sl):
      @pl.when(c == 0)
      def _d0():
        pltpu.sync_copy(accum.at[sl], out_hbm.at[sl, pl.ds(0, 32)])

      @pl.when(c == 1)
      def _d1():
        pltpu.sync_copy(accum.at[sl], out_hbm.at[sl, pl.ds(32, 32)])

    if dr0 * NS == n_dst:
      _drain(pl.ds(s * dr0, dr0))
    else:
      @pl.when(s < NS - 1)
      def _not_last():
        _drain(pl.ds(s * dr0, dr0))

      @pl.when(s == NS - 1)
      def _last():
        _drain(pl.ds((NS - 1) * dr0, dr_last))

  return sc_pass


def kernel(var_feat, constr_feat, edge_index_var_to_constr,
           W_var, b_var, W_constr, b_constr,
           W_v2c, b_v2c, W_c2v, b_c2v, W_score, b_score):
  v = var_feat.shape[0]
  cn = constr_feat.shape[0]
  e = edge_index_var_to_constr.shape[1]

  v_pad = _ceil_to(v + 1, NS * K)
  c_pad = _ceil_to(cn + 1, NS * K)
  e_pad = _ceil_to(e + 1, NS * K * D)
  nchunk_tot = e_pad // K

  eidx = edge_index_var_to_constr.astype(jnp.int32)
  vidx, cidx = eidx[0], eidx[1]
  npad = e_pad - e
  ar = jnp.arange(npad, dtype=jnp.int32)
  sidx_v2c = (2 * jnp.concatenate([vidx, ar % v])).reshape(nchunk_tot, K)
  didx_v2c = jnp.concatenate([cidx, cn + ar % (c_pad - cn)]).reshape(nchunk_tot, K)
  sidx_c2v = (2 * jnp.concatenate([cidx, ar % cn])).reshape(nchunk_tot, K)
  didx_c2v = jnp.concatenate([vidx, v + ar % (v_pad - v)]).reshape(nchunk_tot, K)

  v2c = _make_sc_pass(2 * v, cn, c_pad, nchunk_tot)
  c2v = _make_sc_pass(2 * cn, v, v_pad, nchunk_tot)

  eye2 = jnp.eye(2, dtype=jnp.float32)
  w_var2 = jnp.kron(eye2, W_var)
  w_constr2 = jnp.kron(eye2, W_constr)
  w_v2c2 = jnp.kron(eye2, W_v2c)
  w_c2v2 = jnp.kron(eye2, W_c2v)
  ws2 = jnp.kron(eye2, W_score)
  b_var2 = jnp.tile(b_var, 2)
  b_constr2 = jnp.tile(b_constr, 2)
  b_v2c2 = jnp.tile(b_v2c, 2)
  b_c2v2 = jnp.tile(b_c2v, 2)
  bs2 = jnp.tile(b_score, 2)

  zeros = jnp.zeros((v_pad, 32), jnp.float32)

  h_var = _embed(var_feat.reshape(v // 2, 256), w_var2, b_var2, 5000)
  h_constr = _embed(constr_feat.reshape(cn // 2, 256), w_constr2, b_constr2, cn // 2)

  rounds = 3
  for r in range(rounds):
    msgs_c = v2c(h_var.reshape(2 * v, 32), sidx_v2c, didx_v2c, zeros)
    h_constr = _update(h_constr, msgs_c.reshape(cn // 2, 128),
                       w_v2c2, b_v2c2, cn // 2)
    msgs_v = c2v(h_constr.reshape(2 * cn, 32), sidx_c2v, didx_c2v, zeros)
    if r < rounds - 1:
      h_var = _update(h_var, msgs_v.reshape(v // 2, 128),
                      w_c2v2, b_c2v2, 5000)
    else:
      scores = _score(h_var, msgs_v.reshape(v // 2, 128),
                      w_c2v2, b_c2v2, ws2, bs2, 5000)

  return scores.reshape(-1)
